# Initial kernel scaffold; baseline (speedup 1.0000x reference)
import jax
import jax.numpy as jnp
from jax.experimental import pallas as pl


def kernel(ehr_adj_norm, ddi_adj_norm, w1, b1, w2, b2, w3, b3):
    raise NotImplementedError("write your pallas kernel here")


# speedup vs baseline: 1.0556x; 1.0556x over previous
import jax
import jax.numpy as jnp
from jax.experimental import pallas as pl
from jax.experimental.pallas import tpu as pltpu


def _pad_axis(x, axis, multiple):
    pad = (-x.shape[axis]) % multiple
    if pad == 0:
        return x
    widths = [(0, 0)] * x.ndim
    widths[axis] = (0, pad)
    return jnp.pad(x, widths)


def _branch_kernel(adj_ref, w1_ref, b1_ref, wout_ref, bout_ref, o_ref):
    adj = adj_ref[...]
    h = jnp.dot(adj, w1_ref[...], preferred_element_type=jnp.float32)
    h = jnp.maximum(h + b1_ref[...], 0.0).astype(jnp.bfloat16)
    s = jnp.dot(h, wout_ref[...], preferred_element_type=jnp.float32)
    s = s.astype(jnp.bfloat16)
    o_ref[...] = jnp.dot(adj, s, preferred_element_type=jnp.float32) + bout_ref[...]


def kernel(ehr_adj_norm, ddi_adj_norm, w1, b1, w2, b2, w3, b3):
    bf16 = jnp.bfloat16
    f32 = jnp.float32
    v = ehr_adj_norm.shape[0]
    e = w1.shape[1]

    adj = jnp.stack([ehr_adj_norm, ddi_adj_norm]).astype(bf16)
    adj = _pad_axis(_pad_axis(adj, 1, 8), 2, 128)
    w1p = _pad_axis(_pad_axis(w1.astype(bf16), 0, 128), 1, 128)
    b1p = _pad_axis(b1.reshape(1, e).astype(f32), 1, 128)
    wout = _pad_axis(_pad_axis(jnp.stack([w2, w3]).astype(bf16), 1, 128), 2, 128)
    bout = _pad_axis(jnp.stack([b2, b3]).reshape(2, 1, e).astype(f32), 2, 128)

    vr, vc = adj.shape[1], adj.shape[2]
    ep = w1p.shape[1]
    if vr != vc:
        adj = _pad_axis(adj, 1, vc)
        vr = adj.shape[1]

    out = pl.pallas_call(
        _branch_kernel,
        out_shape=jax.ShapeDtypeStruct((2, vr, ep), f32),
        grid=(2,),
        in_specs=[
            pl.BlockSpec((None, vr, vc), lambda b: (b, 0, 0)),
            pl.BlockSpec((vc, ep), lambda b: (0, 0)),
            pl.BlockSpec((1, ep), lambda b: (0, 0)),
            pl.BlockSpec((None, ep, ep), lambda b: (b, 0, 0)),
            pl.BlockSpec((None, 1, ep), lambda b: (b, 0, 0)),
        ],
        out_specs=pl.BlockSpec((None, vr, ep), lambda b: (b, 0, 0)),
        compiler_params=pltpu.CompilerParams(
            dimension_semantics=("parallel",),
            vmem_limit_bytes=64 * 1024 * 1024),
    )(adj, w1p, b1p, wout, bout)

    out = out[:, :v, :e]
    return out[0], out[1]


# speedup vs baseline: 1.9445x; 1.8421x over previous
import jax
import jax.numpy as jnp
from jax.experimental import pallas as pl
from jax.experimental.pallas import tpu as pltpu

_NCHUNK = 8


def _gcn_kernel(ehr_hbm, ddi_hbm, w1_ref, b1_ref, w2_ref, b2_ref, w3_ref,
                b3_ref, o_ref, adj32, adj_bf, h_scr, sems):
    b = pl.program_id(0)
    v = adj32.shape[0]
    ch = v // _NCHUNK
    f32 = jnp.float32
    bf16 = jnp.bfloat16

    for c in range(_NCHUNK):
        rows = pl.ds(c * ch, ch)

        @pl.when(b == 0)
        def _(rows=rows, c=c):
            pltpu.make_async_copy(ehr_hbm.at[rows], adj32.at[rows],
                                  sems.at[c]).start()

        @pl.when(b == 1)
        def _(rows=rows, c=c):
            pltpu.make_async_copy(ddi_hbm.at[rows], adj32.at[rows],
                                  sems.at[c]).start()

    w1b = w1_ref[...].astype(bf16)
    b1v = b1_ref[...]

    for c in range(_NCHUNK):
        rows = pl.ds(c * ch, ch)
        pltpu.make_async_copy(adj32.at[rows], adj32.at[rows], sems.at[c]).wait()
        ab = adj32[rows, :].astype(bf16)
        adj_bf[rows, :] = ab
        hc = jnp.dot(ab, w1b, preferred_element_type=f32) + b1v
        h_scr[rows, :] = jnp.maximum(hc, 0.0).astype(bf16)

    wout = jnp.where(b == 0, w2_ref[...], w3_ref[...]).astype(bf16)
    bout = jnp.where(b == 0, b2_ref[...], b3_ref[...])
    s = jnp.dot(h_scr[...], wout, preferred_element_type=f32).astype(bf16)
    o_ref[...] = jnp.dot(adj_bf[...], s, preferred_element_type=f32) + bout


def kernel(ehr_adj_norm, ddi_adj_norm, w1, b1, w2, b2, w3, b3):
    f32 = jnp.float32
    v = ehr_adj_norm.shape[0]
    e = w1.shape[1]
    assert v % _NCHUNK == 0 and v % 8 == 0 and e % 128 == 0

    b1r = b1.reshape(1, e)
    b2r = b2.reshape(1, e)
    b3r = b3.reshape(1, e)

    out = pl.pallas_call(
        _gcn_kernel,
        out_shape=jax.ShapeDtypeStruct((2, v, e), f32),
        grid=(2,),
        in_specs=[
            pl.BlockSpec(memory_space=pl.ANY),
            pl.BlockSpec(memory_space=pl.ANY),
            pl.BlockSpec((v, e), lambda b: (0, 0)),
            pl.BlockSpec((1, e), lambda b: (0, 0)),
            pl.BlockSpec((e, e), lambda b: (0, 0)),
            pl.BlockSpec((1, e), lambda b: (0, 0)),
            pl.BlockSpec((e, e), lambda b: (0, 0)),
            pl.BlockSpec((1, e), lambda b: (0, 0)),
        ],
        out_specs=pl.BlockSpec((None, v, e), lambda b: (b, 0, 0)),
        scratch_shapes=[
            pltpu.VMEM((v, v), f32),
            pltpu.VMEM((v, v), jnp.bfloat16),
            pltpu.VMEM((v, e), jnp.bfloat16),
            pltpu.SemaphoreType.DMA((_NCHUNK,)),
        ],
        compiler_params=pltpu.CompilerParams(
            dimension_semantics=("parallel",)),
    )(ehr_adj_norm, ddi_adj_norm, w1, b1r, w2, b2r, w3, b3r)

    return out[0], out[1]
